# baseline (speedup 1.0000x reference)
import jax
import jax.numpy as jnp
from jax.experimental import pallas as pl

_N = 4096
_H1 = 256
_H2 = 64
_BM = 256
_NB = _N // _BM
_NS = 4
_BK = _N // _NS
_ALPHA = 1.0
_BF = jnp.bfloat16


def _dot(a, b):
    return jnp.dot(a, b, preferred_element_type=jnp.float32)


def _dot_t(a, b):
    return jax.lax.dot_general(a, b, (((1,), (1,)), ((), ())),
                               preferred_element_type=jnp.float32)


def _feat_w_kernel(feat_ref, w_ref, out_ref):
    out_ref[...] = _dot(feat_ref[...].astype(_BF), w_ref[...]).astype(_BF)


def _gcn_l1_kernel(a0_ref, a1_ref, a2_ref, a3_ref, u_ref, b1_ref, w2_ref,
                   out_ref):
    acc = _dot(a0_ref[...].astype(_BF), u_ref[0 * _BK:1 * _BK, :])
    acc += _dot(a1_ref[...].astype(_BF), u_ref[1 * _BK:2 * _BK, :])
    acc += _dot(a2_ref[...].astype(_BF), u_ref[2 * _BK:3 * _BK, :])
    acc += _dot(a3_ref[...].astype(_BF), u_ref[3 * _BK:4 * _BK, :])
    x = jax.nn.relu(acc + b1_ref[...])
    out_ref[...] = _dot(x.astype(_BF), w2_ref[...]).astype(_BF)


def _gcn_l2_kernel(a0_ref, a1_ref, a2_ref, a3_ref, v_ref, b2_ref, out_ref):
    acc = _dot(a0_ref[...].astype(_BF), v_ref[0 * _BK:1 * _BK, :])
    acc += _dot(a1_ref[...].astype(_BF), v_ref[1 * _BK:2 * _BK, :])
    acc += _dot(a2_ref[...].astype(_BF), v_ref[2 * _BK:3 * _BK, :])
    acc += _dot(a3_ref[...].astype(_BF), v_ref[3 * _BK:4 * _BK, :])
    out_ref[...] = acc + b2_ref[...]


def _att_w(x, aw1, ab1, aw2t):
    t = jax.nn.relu(_dot(x, aw1) + ab1)
    return jnp.sum(t * aw2t, axis=1, keepdims=True)


def _att_combine(x0, x1, aw1, ab1, aw2t):
    w0 = _att_w(x0, aw1, ab1, aw2t)
    w1 = _att_w(x1, aw1, ab1, aw2t)
    m = jnp.maximum(w0, w1)
    e0 = jnp.exp(w0 - m)
    e1 = jnp.exp(w1 - m)
    inv = 1.0 / (e0 + e1)
    return (e0 * x0 + e1 * x1) * inv


def _z_kernel(h0_ref, h1_ref, aw1_ref, ab1_ref, aw2t_ref, fgw_ref,
              z_ref, y0_ref, y1_ref, s0_ref, s1_ref,
              h0b_ref, h1b_ref, zbh_ref, y0bh_ref, y1bh_ref):
    h0 = h0_ref[...]
    h1 = h1_ref[...]
    z = _att_combine(h0, h1, aw1_ref[...], ab1_ref[...], aw2t_ref[...])
    z_ref[...] = z
    y0 = _dot(h0, fgw_ref[...])
    y1 = _dot(h1, fgw_ref[...])
    y0_ref[...] = y0
    y1_ref[...] = y1
    s0_ref[...] = jnp.sum(y0, axis=0, keepdims=True)
    s1_ref[...] = jnp.sum(y1, axis=0, keepdims=True)
    h0b_ref[...] = h0.astype(_BF)
    h1b_ref[...] = h1.astype(_BF)
    zbh_ref[...] = (0.5 * z).astype(_BF)
    y0bh_ref[...] = (0.5 * y0).astype(_BF)
    y1bh_ref[...] = (0.5 * y1).astype(_BF)


def _cluster_q(x, c, cn2):
    d = (jnp.sum(x * x, axis=1, keepdims=True) - 2.0 * _dot_t(x, c) + cn2)
    q = 1.0 / (1.0 + jnp.maximum(d, 0.0) / _ALPHA)
    q = q ** ((_ALPHA + 1.0) / 2.0)
    return q / jnp.sum(q, axis=1, keepdims=True)


def _fused_tail_kernel(h0b_ref, h1b_ref, z_ref, zbh_ref,
                       y0_ref, y1_ref, y0bh_ref, y1bh_ref,
                       s0_ref, s1_ref, bfg_ref,
                       aw1_ref, ab1_ref, aw2t_ref, c_ref, cn2_ref,
                       wd01_ref, bd01_ref, wd02_ref, bd02_ref,
                       wd11_ref, bd11_ref, wd12_ref, bd12_ref,
                       h_ref, adjz_ref, xz0_ref, xz1_ref, qz_ref, qh_ref):
    i = pl.program_id(0)
    row = pl.ds(i * _BM, _BM)
    bfg = bfg_ref[...]
    aw1 = aw1_ref[...]
    ab1 = ab1_ref[...]
    aw2t = aw2t_ref[...]

    a0c = jnp.tanh(_dot_t(h0b_ref[row, :] * 0.5, h0b_ref[...]))
    hall0 = (_dot(a0c.astype(_BF), y0bh_ref[...]) + 0.5 * s0_ref[...]
             + y1_ref[row, :] + bfg)
    a1c = jnp.tanh(_dot_t(h1b_ref[row, :] * 0.5, h1b_ref[...]))
    hall1 = (y0_ref[row, :] + _dot(a1c.astype(_BF), y1bh_ref[...])
             + 0.5 * s1_ref[...] + bfg)

    hr = _att_combine(hall0, hall1, aw1, ab1, aw2t)
    h_ref[...] = hr

    zr2 = zbh_ref[row, :] * 2.0
    adjz_ref[...] = 0.5 * jnp.tanh(_dot_t(zr2, zbh_ref[...])) + 0.5

    t0 = jax.nn.relu(_dot(zr2, wd01_ref[...]) + bd01_ref[...])
    xz0_ref[...] = _dot(t0.astype(_BF), wd02_ref[...]) + bd02_ref[...]
    t1 = jax.nn.relu(_dot(zr2, wd11_ref[...]) + bd11_ref[...])
    xz1_ref[...] = _dot(t1.astype(_BF), wd12_ref[...]) + bd12_ref[...]

    zr = z_ref[row, :]
    qz_ref[...] = _cluster_q(zr, c_ref[...], cn2_ref[...])
    qh_ref[...] = _cluster_q(hr, c_ref[...], cn2_ref[...])


def _full(shape):
    return pl.BlockSpec(shape, lambda i: tuple(0 for _ in shape))


def _rows(cols, bm=_BM):
    return pl.BlockSpec((bm, cols), lambda i: (i, 0))


def _adj_slices():
    return [pl.BlockSpec((_BM, _BK), lambda i, s=s: (i, s))
            for s in range(_NS)]


def kernel(feat0, feat1, adj0, adj1, params):
    enc = params["enc"]
    dec = params["dec"]
    fgw, fgb = params["fg"]
    aw1, ab1, aw2 = params["att"]
    c = params["cluster"]

    def row2(b):
        return b.reshape(1, -1)

    hidden = []
    for v, (feat, adj) in enumerate(((feat0, adj0), (feat1, adj1))):
        (w1, b1), (w2, b2) = enc[v]
        din = feat.shape[1]
        u = pl.pallas_call(
            _feat_w_kernel,
            grid=(_NB,),
            in_specs=[_rows(din), _full((din, _H1))],
            out_specs=_rows(_H1),
            out_shape=jax.ShapeDtypeStruct((_N, _H1), _BF),
        )(feat, w1.astype(_BF))
        vmat = pl.pallas_call(
            _gcn_l1_kernel,
            grid=(_NB,),
            in_specs=_adj_slices() + [_full((_N, _H1)), _full((1, _H1)),
                                      _full((_H1, _H2))],
            out_specs=_rows(_H2),
            out_shape=jax.ShapeDtypeStruct((_N, _H2), _BF),
        )(adj, adj, adj, adj, u, row2(b1), w2.astype(_BF))
        hp = pl.pallas_call(
            _gcn_l2_kernel,
            grid=(_NB,),
            in_specs=_adj_slices() + [_full((_N, _H2)), _full((1, _H2))],
            out_specs=_rows(_H2),
            out_shape=jax.ShapeDtypeStruct((_N, _H2), jnp.float32),
        )(adj, adj, adj, adj, vmat, row2(b2))
        hidden.append(hp)

    h0, h1 = hidden
    aw2t = aw2.reshape(1, _H2)
    f32 = jnp.float32
    z, y0, y1, s0, s1, h0b, h1b, zbh, y0bh, y1bh = pl.pallas_call(
        _z_kernel,
        out_shape=[jax.ShapeDtypeStruct((_N, _H2), f32)] * 3
        + [jax.ShapeDtypeStruct((1, _H2), f32)] * 2
        + [jax.ShapeDtypeStruct((_N, _H2), _BF)] * 5,
    )(h0, h1, aw1, row2(ab1), aw2t, fgw)

    (wd01, bd01), (wd02, bd02) = dec[0]
    (wd11, bd11), (wd12, bd12) = dec[1]
    dout = wd02.shape[1]
    cn2 = jnp.sum(c * c, axis=1).reshape(1, -1)

    h, adjz, xz0, xz1, qz, qh = pl.pallas_call(
        _fused_tail_kernel,
        grid=(_NB,),
        in_specs=[_full((_N, _H2))] * 8 + [
            _full((1, _H2)), _full((1, _H2)), _full((1, _H2)),
            _full((_H2, _H2)), _full((1, _H2)),
            _full((1, _H2)), _full(c.shape), _full((1, c.shape[0])),
            _full(wd01.shape), _full((1, bd01.shape[0])),
            _full(wd02.shape), _full((1, bd02.shape[0])),
            _full(wd11.shape), _full((1, bd11.shape[0])),
            _full(wd12.shape), _full((1, bd12.shape[0])),
        ],
        out_specs=[_rows(_H2), _rows(_N), _rows(dout), _rows(dout),
                   _rows(c.shape[0]), _rows(c.shape[0])],
        out_shape=[
            jax.ShapeDtypeStruct((_N, _H2), f32),
            jax.ShapeDtypeStruct((_N, _N), f32),
            jax.ShapeDtypeStruct((_N, dout), f32),
            jax.ShapeDtypeStruct((_N, dout), f32),
            jax.ShapeDtypeStruct((_N, c.shape[0]), f32),
            jax.ShapeDtypeStruct((_N, c.shape[0]), f32),
        ],
    )(h0b, h1b, z, zbh, y0, y1, y0bh, y1bh, s0, s1, row2(fgb),
      aw1, row2(ab1), aw2t, c, cn2,
      wd01.astype(_BF), row2(bd01), wd02.astype(_BF), row2(bd02),
      wd11.astype(_BF), row2(bd11), wd12.astype(_BF), row2(bd12))

    return (h, z, adjz, xz0, xz1, qz, qh)


# speedup vs baseline: 1.0429x; 1.0429x over previous
import jax
import jax.numpy as jnp
from jax.experimental import pallas as pl

_N = 4096
_H1 = 256
_H2 = 64
_BM = 256
_NB = _N // _BM
_NS = 4
_BK = _N // _NS
_ALPHA = 1.0


def _dot(a, b):
    return jnp.dot(a, b, preferred_element_type=jnp.float32)


def _dot_t(a, b):
    return jax.lax.dot_general(a, b, (((1,), (1,)), ((), ())),
                               preferred_element_type=jnp.float32)


def _feat_w_kernel(feat_ref, w_ref, out_ref):
    out_ref[...] = _dot(feat_ref[...], w_ref[...])


def _gcn_l1_kernel(a0_ref, a1_ref, a2_ref, a3_ref, u_ref, b1_ref, w2_ref,
                   out_ref):
    acc = _dot(a0_ref[...], u_ref[0 * _BK:1 * _BK, :])
    acc += _dot(a1_ref[...], u_ref[1 * _BK:2 * _BK, :])
    acc += _dot(a2_ref[...], u_ref[2 * _BK:3 * _BK, :])
    acc += _dot(a3_ref[...], u_ref[3 * _BK:4 * _BK, :])
    x = jax.nn.relu(acc + b1_ref[...])
    out_ref[...] = _dot(x, w2_ref[...])


def _gcn_l2_kernel(a0_ref, a1_ref, a2_ref, a3_ref, v_ref, b2_ref, out_ref):
    acc = _dot(a0_ref[...], v_ref[0 * _BK:1 * _BK, :])
    acc += _dot(a1_ref[...], v_ref[1 * _BK:2 * _BK, :])
    acc += _dot(a2_ref[...], v_ref[2 * _BK:3 * _BK, :])
    acc += _dot(a3_ref[...], v_ref[3 * _BK:4 * _BK, :])
    out_ref[...] = acc + b2_ref[...]


def _att_w(x, aw1, ab1, aw2t):
    t = jax.nn.relu(_dot(x, aw1) + ab1)
    return jnp.sum(t * aw2t, axis=1, keepdims=True)


def _att_combine(x0, x1, aw1, ab1, aw2t):
    w0 = _att_w(x0, aw1, ab1, aw2t)
    w1 = _att_w(x1, aw1, ab1, aw2t)
    m = jnp.maximum(w0, w1)
    e0 = jnp.exp(w0 - m)
    e1 = jnp.exp(w1 - m)
    inv = 1.0 / (e0 + e1)
    return (e0 * x0 + e1 * x1) * inv


def _z_kernel(h0_ref, h1_ref, aw1_ref, ab1_ref, aw2t_ref, fgw_ref,
              z_ref, y0_ref, y1_ref, s0_ref, s1_ref, y0h_ref, y1h_ref):
    h0 = h0_ref[...]
    h1 = h1_ref[...]
    z_ref[...] = _att_combine(h0, h1, aw1_ref[...], ab1_ref[...],
                              aw2t_ref[...])
    y0 = _dot(h0, fgw_ref[...])
    y1 = _dot(h1, fgw_ref[...])
    y0_ref[...] = y0
    y1_ref[...] = y1
    s0_ref[...] = jnp.sum(y0, axis=0, keepdims=True)
    s1_ref[...] = jnp.sum(y1, axis=0, keepdims=True)
    y0h_ref[...] = 0.5 * y0
    y1h_ref[...] = 0.5 * y1


def _cluster_q(x, c, cn2):
    d = (jnp.sum(x * x, axis=1, keepdims=True) - 2.0 * _dot_t(x, c) + cn2)
    q = 1.0 / (1.0 + jnp.maximum(d, 0.0) / _ALPHA)
    q = q ** ((_ALPHA + 1.0) / 2.0)
    return q / jnp.sum(q, axis=1, keepdims=True)


def _fused_tail_kernel(h0_ref, h1_ref, z_ref,
                       y0_ref, y1_ref, y0h_ref, y1h_ref,
                       s0_ref, s1_ref, bfg_ref,
                       aw1_ref, ab1_ref, aw2t_ref, c_ref, cn2_ref,
                       wd01_ref, bd01_ref, wd02_ref, bd02_ref,
                       wd11_ref, bd11_ref, wd12_ref, bd12_ref,
                       h_ref, adjz_ref, xz0_ref, xz1_ref, qz_ref, qh_ref):
    i = pl.program_id(0)
    row = pl.ds(i * _BM, _BM)
    bfg = bfg_ref[...]
    aw1 = aw1_ref[...]
    ab1 = ab1_ref[...]
    aw2t = aw2t_ref[...]

    a0c = jnp.tanh(_dot_t(h0_ref[row, :] * 0.5, h0_ref[...]))
    hall0 = (_dot(a0c, y0h_ref[...]) + 0.5 * s0_ref[...]
             + y1_ref[row, :] + bfg)
    a1c = jnp.tanh(_dot_t(h1_ref[row, :] * 0.5, h1_ref[...]))
    hall1 = (y0_ref[row, :] + _dot(a1c, y1h_ref[...])
             + 0.5 * s1_ref[...] + bfg)

    hr = _att_combine(hall0, hall1, aw1, ab1, aw2t)
    h_ref[...] = hr

    zr = z_ref[row, :]
    adjz_ref[...] = 0.5 * jnp.tanh(_dot_t(zr * 0.5, z_ref[...])) + 0.5

    t0 = jax.nn.relu(_dot(zr, wd01_ref[...]) + bd01_ref[...])
    xz0_ref[...] = _dot(t0, wd02_ref[...]) + bd02_ref[...]
    t1 = jax.nn.relu(_dot(zr, wd11_ref[...]) + bd11_ref[...])
    xz1_ref[...] = _dot(t1, wd12_ref[...]) + bd12_ref[...]

    qz_ref[...] = _cluster_q(zr, c_ref[...], cn2_ref[...])
    qh_ref[...] = _cluster_q(hr, c_ref[...], cn2_ref[...])


def _full(shape):
    return pl.BlockSpec(shape, lambda i: tuple(0 for _ in shape))


def _rows(cols, bm=_BM):
    return pl.BlockSpec((bm, cols), lambda i: (i, 0))


def _adj_slices():
    return [pl.BlockSpec((_BM, _BK), lambda i, s=s: (i, s))
            for s in range(_NS)]


def kernel(feat0, feat1, adj0, adj1, params):
    enc = params["enc"]
    dec = params["dec"]
    fgw, fgb = params["fg"]
    aw1, ab1, aw2 = params["att"]
    c = params["cluster"]

    def row2(b):
        return b.reshape(1, -1)

    hidden = []
    for v, (feat, adj) in enumerate(((feat0, adj0), (feat1, adj1))):
        (w1, b1), (w2, b2) = enc[v]
        din = feat.shape[1]
        u = pl.pallas_call(
            _feat_w_kernel,
            grid=(_NB,),
            in_specs=[_rows(din), _full((din, _H1))],
            out_specs=_rows(_H1),
            out_shape=jax.ShapeDtypeStruct((_N, _H1), jnp.float32),
        )(feat, w1)
        vmat = pl.pallas_call(
            _gcn_l1_kernel,
            grid=(_NB,),
            in_specs=_adj_slices() + [_full((_N, _H1)), _full((1, _H1)),
                                      _full((_H1, _H2))],
            out_specs=_rows(_H2),
            out_shape=jax.ShapeDtypeStruct((_N, _H2), jnp.float32),
        )(adj, adj, adj, adj, u, row2(b1), w2)
        hp = pl.pallas_call(
            _gcn_l2_kernel,
            grid=(_NB,),
            in_specs=_adj_slices() + [_full((_N, _H2)), _full((1, _H2))],
            out_specs=_rows(_H2),
            out_shape=jax.ShapeDtypeStruct((_N, _H2), jnp.float32),
        )(adj, adj, adj, adj, vmat, row2(b2))
        hidden.append(hp)

    h0, h1 = hidden
    aw2t = aw2.reshape(1, _H2)
    f32 = jnp.float32
    z, y0, y1, s0, s1, y0h, y1h = pl.pallas_call(
        _z_kernel,
        out_shape=[jax.ShapeDtypeStruct((_N, _H2), f32)] * 3
        + [jax.ShapeDtypeStruct((1, _H2), f32)] * 2
        + [jax.ShapeDtypeStruct((_N, _H2), f32)] * 2,
    )(h0, h1, aw1, row2(ab1), aw2t, fgw)

    (wd01, bd01), (wd02, bd02) = dec[0]
    (wd11, bd11), (wd12, bd12) = dec[1]
    dout = wd02.shape[1]
    cn2 = jnp.sum(c * c, axis=1).reshape(1, -1)

    h, adjz, xz0, xz1, qz, qh = pl.pallas_call(
        _fused_tail_kernel,
        grid=(_NB,),
        in_specs=[_full((_N, _H2))] * 7 + [
            _full((1, _H2)), _full((1, _H2)), _full((1, _H2)),
            _full((_H2, _H2)), _full((1, _H2)),
            _full((1, _H2)), _full(c.shape), _full((1, c.shape[0])),
            _full(wd01.shape), _full((1, bd01.shape[0])),
            _full(wd02.shape), _full((1, bd02.shape[0])),
            _full(wd11.shape), _full((1, bd11.shape[0])),
            _full(wd12.shape), _full((1, bd12.shape[0])),
        ],
        out_specs=[_rows(_H2), _rows(_N), _rows(dout), _rows(dout),
                   _rows(c.shape[0]), _rows(c.shape[0])],
        out_shape=[
            jax.ShapeDtypeStruct((_N, _H2), f32),
            jax.ShapeDtypeStruct((_N, _N), f32),
            jax.ShapeDtypeStruct((_N, dout), f32),
            jax.ShapeDtypeStruct((_N, dout), f32),
            jax.ShapeDtypeStruct((_N, c.shape[0]), f32),
            jax.ShapeDtypeStruct((_N, c.shape[0]), f32),
        ],
    )(h0, h1, z, y0, y1, y0h, y1h, s0, s1, row2(fgb),
      aw1, row2(ab1), aw2t, c, cn2,
      wd01, row2(bd01), wd02, row2(bd02), wd11, row2(bd11), wd12, row2(bd12))

    return (h, z, adjz, xz0, xz1, qz, qh)
